# Initial kernel scaffold; baseline (speedup 1.0000x reference)
import jax
import jax.numpy as jnp
from jax.experimental import pallas as pl


def kernel(x, table):
    raise NotImplementedError("write your pallas kernel here")


# speedup vs baseline: 1.1029x; 1.1029x over previous
import functools

import jax
import jax.numpy as jnp
from jax import lax
from jax.experimental import pallas as pl
from jax.experimental.pallas import tpu as pltpu
from jax.experimental.pallas import tpu_sc as plsc

EMBED_DIM = 32
CHUNK = 1600


@functools.lru_cache(maxsize=None)
def _make_gather(B: int, D: int):
  info = plsc.get_sparse_core_info()
  nc, ns = info.num_cores, info.num_subcores
  nw = nc * ns
  assert B % (nw * CHUNK) == 0
  b_per_w = B // nw
  n_chunks = b_per_w // CHUNK
  mesh = plsc.VectorSubcoreMesh(core_axis_name="c", subcore_axis_name="s")

  @functools.partial(
      pl.kernel,
      mesh=mesh,
      out_type=jax.ShapeDtypeStruct((B, D), jnp.float32),
      compiler_params=pltpu.CompilerParams(use_tc_tiling_on_sc=False),
      scratch_types=[
          pltpu.VMEM((CHUNK,), jnp.int32),
          pltpu.VMEM((CHUNK, D), jnp.float32),
          pltpu.SemaphoreType.DMA,
      ],
  )
  def gather_kernel(idx_hbm, table_hbm, out_hbm, idx_v, rows_v, sem):
    wid = lax.axis_index("s") * nc + lax.axis_index("c")
    base = wid * b_per_w

    def body(i, carry):
      off = base + i * CHUNK
      pltpu.sync_copy(idx_hbm.at[pl.ds(off, CHUNK)], idx_v)
      pltpu.async_copy(table_hbm.at[idx_v], rows_v, sem).wait()
      pltpu.sync_copy(rows_v, out_hbm.at[pl.ds(off, CHUNK)])
      return carry

    lax.fori_loop(0, n_chunks, body, 0)

  return gather_kernel


def kernel(x, table):
  b0, b1 = x.shape
  flat_idx = x.reshape(b0 * b1).astype(jnp.int32)
  out = _make_gather(b0 * b1, table.shape[1])(flat_idx, table)
  return out.reshape(b0, b1, table.shape[1])


# speedup vs baseline: 1.1129x; 1.0090x over previous
import functools

import jax
import jax.numpy as jnp
from jax import lax
from jax.experimental import pallas as pl
from jax.experimental.pallas import tpu as pltpu
from jax.experimental.pallas import tpu_sc as plsc

EMBED_DIM = 32
CHUNK = 1600


@functools.lru_cache(maxsize=None)
def _make_gather(B: int, D: int):
  info = plsc.get_sparse_core_info()
  nc, ns = info.num_cores, info.num_subcores
  nw = nc * ns
  assert B % (nw * CHUNK) == 0
  b_per_w = B // nw
  n_chunks = b_per_w // CHUNK
  mesh = plsc.VectorSubcoreMesh(core_axis_name="c", subcore_axis_name="s")

  @functools.partial(
      pl.kernel,
      mesh=mesh,
      out_type=jax.ShapeDtypeStruct((B, D), jnp.float32),
      compiler_params=pltpu.CompilerParams(use_tc_tiling_on_sc=False),
      scratch_types=[
          pltpu.VMEM((b_per_w,), jnp.int32),
          pltpu.VMEM((2, CHUNK, D), jnp.float32),
          pltpu.SemaphoreType.DMA,
          pltpu.SemaphoreType.DMA,
          pltpu.SemaphoreType.DMA,
          pltpu.SemaphoreType.DMA,
      ],
  )
  def gather_kernel(idx_hbm, table_hbm, out_hbm, idx_v, rows_v,
                    gsem0, gsem1, ssem0, ssem1):
    wid = lax.axis_index("s") * nc + lax.axis_index("c")
    base = wid * b_per_w
    gsems = (gsem0, gsem1)
    ssems = (ssem0, ssem1)

    pltpu.sync_copy(idx_hbm.at[pl.ds(base, b_per_w)], idx_v)

    def start_gather(i, b):
      pltpu.async_copy(
          table_hbm.at[idx_v.at[pl.ds(i * CHUNK, CHUNK)]],
          rows_v.at[b], gsems[b])

    def wait_gather(i, b):
      pltpu.make_async_copy(
          table_hbm.at[idx_v.at[pl.ds(i * CHUNK, CHUNK)]],
          rows_v.at[b], gsems[b]).wait()

    def start_store(i, b):
      pltpu.async_copy(
          rows_v.at[b], out_hbm.at[pl.ds(base + i * CHUNK, CHUNK)], ssems[b])

    def wait_store(i, b):
      pltpu.make_async_copy(
          rows_v.at[b], out_hbm.at[pl.ds(base + i * CHUNK, CHUNK)],
          ssems[b]).wait()

    start_gather(0, 0)
    for i in range(n_chunks):
      b = i % 2
      if i + 1 < n_chunks:
        if i >= 1:
          wait_store(i - 1, 1 - b)
        start_gather(i + 1, 1 - b)
      wait_gather(i, b)
      start_store(i, b)
    wait_store(n_chunks - 2, n_chunks % 2)
    wait_store(n_chunks - 1, (n_chunks - 1) % 2)

  return gather_kernel


def kernel(x, table):
  b0, b1 = x.shape
  flat_idx = x.reshape(b0 * b1).astype(jnp.int32)
  out = _make_gather(b0 * b1, table.shape[1])(flat_idx, table)
  return out.reshape(b0, b1, table.shape[1])


# speedup vs baseline: 1.3549x; 1.2175x over previous
import functools

import jax
import jax.numpy as jnp
from jax import lax
from jax.experimental import pallas as pl
from jax.experimental.pallas import tpu as pltpu
from jax.experimental.pallas import tpu_sc as plsc

EMBED_DIM = 32
CHUNK = 512


@functools.lru_cache(maxsize=None)
def _make_gather(S: int, Bt: int, D: int):
  info = plsc.get_sparse_core_info()
  nc, ns = info.num_cores, info.num_subcores
  nw = nc * ns
  assert Bt % (nw * 16) == 0 and D == EMBED_DIM
  assert CHUNK * nw == Bt
  n_tiles = CHUNK // 16
  mesh = plsc.VectorSubcoreMesh(core_axis_name="c", subcore_axis_name="s")

  @functools.partial(
      pl.kernel,
      mesh=mesh,
      out_type=jax.ShapeDtypeStruct((S, D, Bt), jnp.float32),
      compiler_params=pltpu.CompilerParams(needs_layout_passes=False),
      scratch_types=[
          pltpu.VMEM((CHUNK,), jnp.int32),
          pltpu.VMEM((CHUNK,), jnp.int32),
          pltpu.VMEM((CHUNK, 128), jnp.float32),
          pltpu.VMEM((D, CHUNK), jnp.float32),
          pltpu.SemaphoreType.DMA,
      ],
  )
  def gather_kernel(xt_hbm, t128_hbm, out_hbm, idx_v, idxg_v, rows_v,
                    tbuf, sem):
    wid = lax.axis_index("s") * nc + lax.axis_index("c")
    b0 = wid * CHUNK
    lane = lax.iota(jnp.int32, 16)

    def seq_body(s, carry):
      pltpu.sync_copy(xt_hbm.at[s, pl.ds(b0, CHUNK)], idx_v)

      def shift_body(t, carry):
        idxg_v[pl.ds(t * 16, 16)] = (
            lax.shift_right_logical(idx_v[pl.ds(t * 16, 16)], 2))
        return carry

      lax.fori_loop(0, n_tiles, shift_body, 0, unroll=4)

      pltpu.async_copy(t128_hbm.at[idxg_v], rows_v, sem).wait()

      def sel_body(t, carry):
        c0 = t * 16
        v = idx_v[pl.ds(c0, 16)]
        col0 = lax.shift_left(lax.bitwise_and(v, 3), 5)
        row = c0 + lane
        for d in range(D):
          tbuf[d, pl.ds(c0, 16)] = plsc.load_gather(
              rows_v, [row, col0 + d])
        return carry

      lax.fori_loop(0, n_tiles, sel_body, 0)

      pltpu.sync_copy(tbuf, out_hbm.at[s, :, pl.ds(b0, CHUNK)])
      return carry

    lax.fori_loop(0, S, seq_body, 0)

  return gather_kernel


def kernel(x, table):
  b, s = x.shape
  d = table.shape[1]
  t128 = table.reshape(table.shape[0] * d // 128, 128)
  xt = x.T
  out_t = _make_gather(s, b, d)(xt, t128)
  return out_t.transpose(2, 0, 1)


# speedup vs baseline: 1.6668x; 1.2303x over previous
import functools

import jax
import jax.numpy as jnp
from jax import lax
from jax.experimental import pallas as pl
from jax.experimental.pallas import tpu as pltpu
from jax.experimental.pallas import tpu_sc as plsc

EMBED_DIM = 32
CHUNK = 512


@functools.lru_cache(maxsize=None)
def _make_gather(S: int, Bt: int, D: int):
  info = plsc.get_sparse_core_info()
  nc, ns = info.num_cores, info.num_subcores
  nw = nc * ns
  assert Bt % (nw * 16) == 0 and D == EMBED_DIM
  assert CHUNK * nw == Bt
  n_tiles = CHUNK // 16
  mesh = plsc.VectorSubcoreMesh(core_axis_name="c", subcore_axis_name="s")

  @functools.partial(
      pl.kernel,
      mesh=mesh,
      out_type=jax.ShapeDtypeStruct((S, D, Bt), jnp.float32),
      compiler_params=pltpu.CompilerParams(needs_layout_passes=False),
      scratch_types=[
          pltpu.VMEM((CHUNK,), jnp.int32),
          pltpu.VMEM((CHUNK,), jnp.int32),
          pltpu.VMEM((CHUNK, 128), jnp.float32),
          pltpu.VMEM((D, CHUNK), jnp.float32),
          pltpu.SemaphoreType.DMA,
      ],
  )
  def gather_kernel(xt_hbm, t128_hbm, out_hbm, idx_v, idxg_v, rows_v,
                    tbuf, sem):
    wid = lax.axis_index("s") * nc + lax.axis_index("c")
    b0 = wid * CHUNK
    lane = lax.iota(jnp.int32, 16)

    def seq_body(s, carry):
      pltpu.sync_copy(xt_hbm.at[s, pl.ds(b0, CHUNK)], idx_v)

      @plsc.parallel_loop(0, n_tiles, 1, unroll=4)
      def _shift(t):
        idxg_v[pl.ds(t * 16, 16)] = (
            lax.shift_right_logical(idx_v[pl.ds(t * 16, 16)], 2))

      pltpu.async_copy(t128_hbm.at[idxg_v], rows_v, sem).wait()

      @plsc.parallel_loop(0, n_tiles, 1, unroll=2)
      def _select(t):
        c0 = t * 16
        v = idx_v[pl.ds(c0, 16)]
        col0 = lax.shift_left(lax.bitwise_and(v, 3), 5)
        row = c0 + lane
        for d in range(D):
          tbuf[d, pl.ds(c0, 16)] = plsc.load_gather(
              rows_v, [row, col0 + d])

      pltpu.sync_copy(tbuf, out_hbm.at[s, :, pl.ds(b0, CHUNK)])
      return carry

    lax.fori_loop(0, S, seq_body, 0)

  return gather_kernel


def kernel(x, table):
  b, s = x.shape
  d = table.shape[1]
  t128 = table.reshape(table.shape[0] * d // 128, 128)
  xt = x.T
  out_t = _make_gather(s, b, d)(xt, t128)
  return out_t.transpose(2, 0, 1)
